# baseline (speedup 1.0000x reference)
import functools

import jax
import jax.numpy as jnp
from jax import lax
from jax.experimental import pallas as pl
from jax.experimental.pallas import tpu as pltpu
from jax.experimental.pallas import tpu_sc as plsc

NC = 2
NS = 16
NW = NC * NS
CHUNK = 40
D = 300
DPAD = 384
LANES = 16


def _body(table_hbm, idx_hbm, out_hbm, idx_v, rows_v, pack_v, sem):
    nch = idx_hbm.shape[1]
    seq = out_hbm.shape[1]
    cpb = seq // CHUNK
    bpw = nch // cpb
    wid = lax.axis_index("s") * NC + lax.axis_index("c")
    pltpu.sync_copy(idx_hbm.at[wid], idx_v)

    tail_idx = jax.lax.iota(jnp.int32, LANES) + (D // LANES) * LANES
    tail_mask = jax.lax.iota(jnp.int32, LANES) < (D % LANES)

    def repack_row(r, carry):
        for k in range(D // LANES):
            pack_v[r, pl.ds(k * LANES, LANES)] = rows_v[r, pl.ds(k * LANES, LANES)]
        tail = rows_v[r, pl.ds((D // LANES) * LANES, LANES)]
        row_idx = jnp.full((LANES,), r, dtype=jnp.int32)
        plsc.store_scatter(pack_v, [row_idx, tail_idx], tail, mask=tail_mask)
        return carry

    def step(j, carry):
        pltpu.async_copy(table_hbm.at[idx_v.at[j]], rows_v, sem).wait()
        lax.fori_loop(0, CHUNK, repack_row, 0)
        b = wid * bpw + lax.div(j, cpb)
        s0 = lax.rem(j, cpb) * CHUNK
        pltpu.sync_copy(pack_v, out_hbm.at[b, pl.ds(s0, CHUNK)])
        return carry

    lax.fori_loop(0, nch, step, 0)


@functools.lru_cache(maxsize=None)
def _make(batch, seq, nch):
    mesh = plsc.VectorSubcoreMesh(core_axis_name="c", subcore_axis_name="s")
    return pl.kernel(
        _body,
        out_type=jax.ShapeDtypeStruct((batch, seq, D), jnp.float32),
        mesh=mesh,
        scratch_types=[
            pltpu.VMEM((nch, CHUNK), jnp.int32),
            pltpu.VMEM((CHUNK, DPAD), jnp.float32),
            pltpu.VMEM((CHUNK, D), jnp.float32),
            pltpu.SemaphoreType.DMA,
        ],
        compiler_params=pltpu.CompilerParams(needs_layout_passes=False),
    )


def kernel(word_input, table):
    b, s = word_input.shape
    vocab, d = table.shape
    idx = word_input.reshape(-1).astype(jnp.int32)
    total = b * s
    per_w = total // NW
    nch = per_w // CHUNK
    idx3 = idx.reshape(NW, nch, CHUNK)
    tab_pad = jnp.pad(table, ((0, 0), (0, DPAD - d)))
    return _make(b, s, nch)(tab_pad, idx3)


# speedup vs baseline: 1.7693x; 1.7693x over previous
import functools

import jax
import jax.numpy as jnp
from jax import lax
from jax.experimental import pallas as pl
from jax.experimental.pallas import tpu as pltpu
from jax.experimental.pallas import tpu_sc as plsc

NC = 2
NS = 16
NW = NC * NS
CHUNK = 40
D = 300
DPAD = 384
LANES = 16


def _pad_body(x_ref, o_ref):
    o_ref[:, :D] = x_ref[...]


@functools.lru_cache(maxsize=None)
def _make_pad(vocab):
    blk = 4000
    return pl.pallas_call(
        _pad_body,
        out_shape=jax.ShapeDtypeStruct((vocab, DPAD), jnp.float32),
        grid=(vocab // blk,),
        in_specs=[pl.BlockSpec((blk, D), lambda i: (i, 0))],
        out_specs=pl.BlockSpec((blk, DPAD), lambda i: (i, 0)),
    )


def _body(table_hbm, idx_hbm, out_hbm, idx_v, rows_v, pack_v, gsem, osem):
    nch = idx_hbm.shape[1]
    seq = out_hbm.shape[1]
    cpb = seq // CHUNK
    bpw = nch // cpb
    wid = lax.axis_index("s") * NC + lax.axis_index("c")
    pltpu.sync_copy(idx_hbm.at[wid], idx_v)

    tail_idx = jax.lax.iota(jnp.int32, LANES) + (D // LANES) * LANES
    tail_mask = jax.lax.iota(jnp.int32, LANES) < (D % LANES)

    def out_slice(j):
        b = wid * bpw + lax.div(j, cpb)
        s0 = lax.rem(j, cpb) * CHUNK
        return out_hbm.at[b, pl.ds(s0, CHUNK)]

    def fire_gather(j, slot):
        pltpu.async_copy(table_hbm.at[idx_v.at[j]], rows_v.at[slot],
                         gsem.at[slot])

    def wait_gather(slot):
        pltpu.make_async_copy(table_hbm.at[pl.ds(0, CHUNK)], rows_v.at[slot],
                              gsem.at[slot]).wait()

    def fire_out(j, slot):
        pltpu.async_copy(pack_v.at[slot], out_slice(j), osem.at[slot])

    def wait_out(slot):
        pltpu.make_async_copy(pack_v.at[slot], out_hbm.at[0, pl.ds(0, CHUNK)],
                              osem.at[slot]).wait()

    def repack(slot):
        def row(r, carry):
            for k in range(D // LANES):
                pack_v[slot, r, pl.ds(k * LANES, LANES)] = (
                    rows_v[slot, r, pl.ds(k * LANES, LANES)])
            tail = rows_v[slot, r, pl.ds((D // LANES) * LANES, LANES)]
            row_idx = jnp.full((LANES,), slot, dtype=jnp.int32)
            r_idx = jnp.full((LANES,), r, dtype=jnp.int32)
            plsc.store_scatter(pack_v, [row_idx, r_idx, tail_idx], tail,
                               mask=tail_mask)
            return carry
        lax.fori_loop(0, CHUNK, row, 0)

    fire_gather(0, 0)
    fire_gather(1, 1)

    def step(j, carry):
        slot = lax.rem(j, 2)

        @pl.when(j >= 2)
        def _():
            wait_out(slot)
        wait_gather(slot)
        repack(slot)

        @pl.when(j + 2 < nch)
        def _():
            fire_gather(j + 2, slot)
        fire_out(j, slot)
        return carry

    lax.fori_loop(0, nch, step, 0)
    wait_out(0)
    wait_out(1)


@functools.lru_cache(maxsize=None)
def _make(batch, seq, nch):
    mesh = plsc.VectorSubcoreMesh(core_axis_name="c", subcore_axis_name="s")
    return pl.kernel(
        _body,
        out_type=jax.ShapeDtypeStruct((batch, seq, D), jnp.float32),
        mesh=mesh,
        scratch_types=[
            pltpu.VMEM((nch, CHUNK), jnp.int32),
            pltpu.VMEM((2, CHUNK, DPAD), jnp.float32),
            pltpu.VMEM((2, CHUNK, D), jnp.float32),
            pltpu.SemaphoreType.DMA((2,)),
            pltpu.SemaphoreType.DMA((2,)),
        ],
        compiler_params=pltpu.CompilerParams(needs_layout_passes=False),
    )


def kernel(word_input, table):
    b, s = word_input.shape
    vocab, d = table.shape
    idx = word_input.reshape(-1).astype(jnp.int32)
    total = b * s
    per_w = total // NW
    nch = per_w // CHUNK
    idx3 = idx.reshape(NW, nch, CHUNK)
    tab_pad = _make_pad(vocab)(table)
    return _make(b, s, nch)(tab_pad, idx3)
